# baseline (speedup 1.0000x reference)
import jax
import jax.numpy as jnp
from jax import lax
from jax.experimental import pallas as pl
from jax.experimental.pallas import tpu as pltpu
from jax.experimental.pallas import tpu_sc as plsc

N_NODES = 10000
D = 128
N_EDGES = 320000

NC = 2
NS = 16
NW = NC * NS
CHUNK = 128
FCHUNK = 32
FN_CHUNKS = 10240 // FCHUNK
K0 = 184
K1 = 456
E_PAD = 327680
EPW = E_PAD // NW
N_CHUNKS = EPW // CHUNK
N_PAD = 10240
ROWS_PER_TILE = N_PAD // NS
ZROWS = 128
BLK = 1024


def _linear_body(x_ref, w_ref, b_ref, o_ref):
    o_ref[...] = lax.dot_general(
        x_ref[...], w_ref[...], (((1,), (1,)), ((), ())),
        preferred_element_type=jnp.float32) + b_ref[...]


def _tc_linear(x, W, b):
    return pl.pallas_call(
        _linear_body,
        grid=(10,),
        in_specs=[
            pl.BlockSpec((1000, D), lambda i: (i, 0)),
            pl.BlockSpec((D, D), lambda i: (0, 0)),
            pl.BlockSpec((1, D), lambda i: (0, 0)),
        ],
        out_specs=pl.BlockSpec((1000, D), lambda i: (i, 0)),
        out_shape=jax.ShapeDtypeStruct((N_NODES, D), jnp.float32),
    )(x, W, b.reshape(1, D))


def _combine_body(a0_ref, a1_ref, n0_ref, n1_ref, x_ref, w_ref, b_ref, o_ref):
    norm = n0_ref[...][:, 0:1] + n1_ref[...][:, 0:1]
    acc = a0_ref[...] + a1_ref[...]
    selfh = lax.dot_general(
        x_ref[...], w_ref[...], (((1,), (1,)), ((), ())),
        preferred_element_type=jnp.float32) + b_ref[...]
    o_ref[...] = acc / norm + selfh


def _tc_combine(a0, a1, n0, n1, x_pad, W_self, b_self):
    return pl.pallas_call(
        _combine_body,
        grid=(10,),
        in_specs=[
            pl.BlockSpec((BLK, D), lambda i: (i, 0)),
            pl.BlockSpec((BLK, D), lambda i: (i, 0)),
            pl.BlockSpec((BLK, D), lambda i: (i, 0)),
            pl.BlockSpec((BLK, D), lambda i: (i, 0)),
            pl.BlockSpec((BLK, D), lambda i: (i, 0)),
            pl.BlockSpec((D, D), lambda i: (0, 0)),
            pl.BlockSpec((1, D), lambda i: (0, 0)),
        ],
        out_specs=pl.BlockSpec((BLK, D), lambda i: (i, 0)),
        out_shape=jax.ShapeDtypeStruct((N_PAD, D), jnp.float32),
    )(a0, a1, n0, n1, x_pad, W_self, b_self.reshape(1, D))


NBUF = 4


def _sc_feat_body(h_hbm, src_hbm, dst_hbm, adj_hbm,
                  acc_out,
                  acc_sh,
                  srcv0, dstv0, adjv0, srcv1, dstv1, adjv1,
                  srcv2, dstv2, adjv2, srcv3, dstv3, adjv3,
                  rows0, rows1, rows2, rows3,
                  semi0, semi1, semi2, semi3,
                  semg0, semg1, semg2, semg3,
                  sems0, sems1, sems2, sems3):
    c = lax.axis_index("c")
    s = lax.axis_index("s")
    myk = jnp.where(c == 0, K0, K1)
    ebase = jnp.where(c == 0, s * K0, (NS * K0) + s * K1) * FCHUNK

    srcv = [srcv0, srcv1, srcv2, srcv3]
    dstv = [dstv0, dstv1, dstv2, dstv3]
    adjv = [adjv0, adjv1, adjv2, adjv3]
    rows = [rows0, rows1, rows2, rows3]
    semi = [semi0, semi1, semi2, semi3]
    semg = [semg0, semg1, semg2, semg3]
    sems = [sems0, sems1, sems2, sems3]

    def idx_start(ci, b):
        base = ebase + ci * FCHUNK
        pltpu.async_copy(src_hbm.at[pl.ds(base, FCHUNK)], srcv[b], semi[b])
        pltpu.async_copy(dst_hbm.at[pl.ds(base, FCHUNK)], dstv[b], semi[b])
        pltpu.async_copy(adj_hbm.at[pl.ds(base, FCHUNK)], adjv[b], semi[b])

    def idx_wait(b):
        pltpu.make_async_copy(src_hbm.at[pl.ds(0, FCHUNK)], srcv[b], semi[b]).wait()
        pltpu.make_async_copy(dst_hbm.at[pl.ds(0, FCHUNK)], dstv[b], semi[b]).wait()
        pltpu.make_async_copy(adj_hbm.at[pl.ds(0, FCHUNK)], adjv[b], semi[b]).wait()

    def gather_start(b):
        pltpu.async_copy(h_hbm.at[srcv[b]], rows[b], semg[b])

    def gather_wait(b):
        pltpu.make_async_copy(h_hbm.at[srcv[b]], rows[b], semg[b]).wait()

    def scatter_start(b):
        pltpu.async_copy(rows[b], acc_sh.at[dstv[b]], sems[b], add=True)

    def scatter_wait(b):
        pltpu.make_async_copy(rows[b], acc_sh.at[dstv[b]], sems[b]).wait()

    def scale(b):
        def _scale(g, _):
            av = adjv[b][pl.ds(g * 16, 16)]
            for j in range(16):
                a = jnp.full((16,), av[j])
                e = g * 16 + j
                for k in range(D // 16):
                    rows[b][e, pl.ds(k * 16, 16)] = (
                        rows[b][e, pl.ds(k * 16, 16)] * a)
            return 0
        lax.fori_loop(0, FCHUNK // 16, _scale, 0)

    z16 = jnp.zeros((16,), jnp.float32)

    def _zero_bufs(r, _):
        for k in range(D // 16):
            for b in range(NBUF):
                rows[b][r, pl.ds(k * 16, 16)] = z16
        return 0
    lax.fori_loop(0, FCHUNK, _zero_bufs, 0)

    row0 = s * ROWS_PER_TILE
    for k in range(ROWS_PER_TILE // FCHUNK):
        pltpu.sync_copy(rows0, acc_sh.at[pl.ds(row0 + k * FCHUNK, FCHUNK)])

    plsc.subcore_barrier()

    for b in range(NBUF):
        idx_start(b, b)
    for b in range(NBUF):
        idx_wait(b)
        gather_start(b)

    def _quad(k, _):
        for b in range(NBUF):
            cb = NBUF * k + b
            gather_wait(b)

            pb = (b + NBUF - 1) % NBUF

            @pl.when(cb >= 1)
            def _():
                scatter_wait(pb)

                @pl.when(cb + NBUF - 1 < myk)
                def _():
                    idx_start(cb + NBUF - 1, pb)
                    idx_wait(pb)
                    gather_start(pb)

            scale(b)
            scatter_start(b)
        return 0

    lax.fori_loop(0, myk // NBUF, _quad, 0)
    scatter_wait(NBUF - 1)

    plsc.subcore_barrier()

    for k in range(ROWS_PER_TILE // FCHUNK):
        r = row0 + k * FCHUNK
        pltpu.sync_copy(acc_sh.at[pl.ds(r, FCHUNK)], rows0)
        pltpu.sync_copy(rows0, acc_out.at[c, pl.ds(r, FCHUNK)])


def _sc_feat(h, src, dst, adj):
    mesh = plsc.VectorSubcoreMesh(core_axis_name="c", subcore_axis_name="s")
    idx_t = [pltpu.VMEM((FCHUNK,), jnp.int32),
             pltpu.VMEM((FCHUNK,), jnp.int32),
             pltpu.VMEM((FCHUNK,), jnp.float32)]
    f = pl.kernel(
        _sc_feat_body,
        out_type=jax.ShapeDtypeStruct((NC, N_PAD, D), jnp.float32),
        mesh=mesh,
        scratch_types=(
            [pltpu.VMEM_SHARED((N_PAD, D), jnp.float32)]
            + idx_t * NBUF
            + [pltpu.VMEM((FCHUNK, D), jnp.float32)] * NBUF
            + [pltpu.SemaphoreType.DMA] * (3 * NBUF)
        ),
    )
    return f(h, src, dst, adj)


def _sc_norm_body(dst_hbm, adj_hbm, norm_out, norm_sh,
                  dstv0, adjv0, dstv1, adjv1, nr0, nr1,
                  semi0, semi1, sems0, sems1):
    c = lax.axis_index("c")
    s = lax.axis_index("s")
    wid = c * NS + s
    ebase = wid * EPW

    def idx_start(ci, dv, av, sem):
        base = ebase + ci * FCHUNK
        pltpu.async_copy(dst_hbm.at[pl.ds(base, FCHUNK)], dv, sem)
        pltpu.async_copy(adj_hbm.at[pl.ds(base, FCHUNK)], av, sem)

    def idx_wait(dv, av, sem):
        pltpu.make_async_copy(dst_hbm.at[pl.ds(0, FCHUNK)], dv, sem).wait()
        pltpu.make_async_copy(adj_hbm.at[pl.ds(0, FCHUNK)], av, sem).wait()

    def fill(nr, av_ref):
        def _f(g, _):
            av = av_ref[pl.ds(g * 16, 16)]
            for j in range(16):
                nr[g * 16 + j, pl.ds(0, 16)] = jnp.full((16,), av[j])
            return 0
        lax.fori_loop(0, FCHUNK // 16, _f, 0)

    z16 = jnp.zeros((16,), jnp.float32)

    def _zero_bufs(r, _):
        for k in range(D // 16):
            nr0[r, pl.ds(k * 16, 16)] = z16
            nr1[r, pl.ds(k * 16, 16)] = z16
        return 0
    lax.fori_loop(0, FCHUNK, _zero_bufs, 0)

    row0 = s * ROWS_PER_TILE
    for k in range(ROWS_PER_TILE // FCHUNK):
        pltpu.sync_copy(nr0, norm_sh.at[pl.ds(row0 + k * FCHUNK, FCHUNK)])

    plsc.subcore_barrier()

    idx_start(0, dstv0, adjv0, semi0)
    idx_wait(dstv0, adjv0, semi0)
    fill(nr0, adjv0)
    idx_start(1, dstv1, adjv1, semi1)

    def _pair(k, _):
        c0 = 2 * k
        d0 = pltpu.async_copy(nr0, norm_sh.at[dstv0], sems0, add=True)
        idx_wait(dstv1, adjv1, semi1)
        fill(nr1, adjv1)
        d1 = pltpu.async_copy(nr1, norm_sh.at[dstv1], sems1, add=True)
        d0.wait()

        @pl.when(c0 + 2 < FN_CHUNKS)
        def _():
            idx_start(c0 + 2, dstv0, adjv0, semi0)
            idx_wait(dstv0, adjv0, semi0)
            fill(nr0, adjv0)

        d1.wait()

        @pl.when(c0 + 3 < FN_CHUNKS)
        def _():
            idx_start(c0 + 3, dstv1, adjv1, semi1)
        return 0

    lax.fori_loop(0, FN_CHUNKS // 2, _pair, 0)

    plsc.subcore_barrier()

    for k in range(ROWS_PER_TILE // FCHUNK):
        r = row0 + k * FCHUNK
        pltpu.sync_copy(norm_sh.at[pl.ds(r, FCHUNK)], nr0)
        pltpu.sync_copy(nr0, norm_out.at[c, pl.ds(r, FCHUNK)])


def _sc_norm(dst, adj):
    mesh = plsc.VectorSubcoreMesh(core_axis_name="c", subcore_axis_name="s")
    f = pl.kernel(
        _sc_norm_body,
        out_type=jax.ShapeDtypeStruct((NC, N_PAD, D), jnp.float32),
        mesh=mesh,
        scratch_types=[
            pltpu.VMEM_SHARED((N_PAD, D), jnp.float32),
            pltpu.VMEM((FCHUNK,), jnp.int32),
            pltpu.VMEM((FCHUNK,), jnp.float32),
            pltpu.VMEM((FCHUNK,), jnp.int32),
            pltpu.VMEM((FCHUNK,), jnp.float32),
            pltpu.VMEM((FCHUNK, D), jnp.float32),
            pltpu.VMEM((FCHUNK, D), jnp.float32),
            pltpu.SemaphoreType.DMA,
            pltpu.SemaphoreType.DMA,
            pltpu.SemaphoreType.DMA,
            pltpu.SemaphoreType.DMA,
        ],
    )
    return f(dst, adj)


def kernel(node_feat, edge_index, adj_values, W, b, W_self, b_self):
    x = node_feat[0]
    dst = edge_index[0].astype(jnp.int32)
    src = edge_index[1].astype(jnp.int32)
    pad = E_PAD - N_EDGES
    src_p = jnp.concatenate([src, jnp.zeros((pad,), jnp.int32)])
    dst_p = jnp.concatenate([dst, jnp.zeros((pad,), jnp.int32)])
    adj_p = jnp.concatenate([adj_values, jnp.zeros((pad,), jnp.float32)])
    x_pad = jnp.concatenate(
        [x, jnp.zeros((N_PAD - N_NODES, D), jnp.float32)], axis=0)

    h = _tc_linear(x, W, b)
    acc = _sc_feat(h, src_p, dst_p, adj_p)
    normp = _sc_norm(dst_p, adj_p)
    out = _tc_combine(acc[0], acc[1], normp[0], normp[1], x_pad, W_self, b_self)
    return out[:N_NODES][None]


# speedup vs baseline: 1.2207x; 1.2207x over previous
import jax
import jax.numpy as jnp
from jax import lax
from jax.experimental import pallas as pl
from jax.experimental.pallas import tpu as pltpu
from jax.experimental.pallas import tpu_sc as plsc

N_NODES = 10000
D = 128
N_EDGES = 320000

NC = 2
NS = 16
NW = NC * NS
CHUNK = 128
FCHUNK = 32
FN_CHUNKS = 10240 // FCHUNK
K0 = 456
K1 = 184
E_PAD = 327680
EPW = E_PAD // NW
N_CHUNKS = EPW // CHUNK
N_PAD = 10240
ROWS_PER_TILE = N_PAD // NS
ZROWS = 128
BLK = 1024


def _linear_body(x_ref, w_ref, b_ref, o_ref):
    o_ref[...] = lax.dot_general(
        x_ref[...], w_ref[...], (((1,), (1,)), ((), ())),
        preferred_element_type=jnp.float32) + b_ref[...]


def _tc_linear(x, W, b):
    return pl.pallas_call(
        _linear_body,
        grid=(10,),
        in_specs=[
            pl.BlockSpec((1000, D), lambda i: (i, 0)),
            pl.BlockSpec((D, D), lambda i: (0, 0)),
            pl.BlockSpec((1, D), lambda i: (0, 0)),
        ],
        out_specs=pl.BlockSpec((1000, D), lambda i: (i, 0)),
        out_shape=jax.ShapeDtypeStruct((N_NODES, D), jnp.float32),
    )(x, W, b.reshape(1, D))


def _combine_body(a0_ref, a1_ref, n0_ref, n1_ref, x_ref, w_ref, b_ref, o_ref):
    norm = n0_ref[...][:, 0:1] + n1_ref[...][:, 0:1]
    acc = a0_ref[...] + a1_ref[...]
    selfh = lax.dot_general(
        x_ref[...], w_ref[...], (((1,), (1,)), ((), ())),
        preferred_element_type=jnp.float32) + b_ref[...]
    o_ref[...] = acc / norm + selfh


def _tc_combine(a0, a1, n0, n1, x_pad, W_self, b_self):
    return pl.pallas_call(
        _combine_body,
        grid=(10,),
        in_specs=[
            pl.BlockSpec((BLK, D), lambda i: (i, 0)),
            pl.BlockSpec((BLK, D), lambda i: (i, 0)),
            pl.BlockSpec((BLK, D), lambda i: (i, 0)),
            pl.BlockSpec((BLK, D), lambda i: (i, 0)),
            pl.BlockSpec((BLK, D), lambda i: (i, 0)),
            pl.BlockSpec((D, D), lambda i: (0, 0)),
            pl.BlockSpec((1, D), lambda i: (0, 0)),
        ],
        out_specs=pl.BlockSpec((BLK, D), lambda i: (i, 0)),
        out_shape=jax.ShapeDtypeStruct((N_PAD, D), jnp.float32),
    )(a0, a1, n0, n1, x_pad, W_self, b_self.reshape(1, D))


NBUF = 4


def _sc_feat_body(h_hbm, src_hbm, dst_hbm, adj_hbm,
                  acc_out,
                  acc_sh,
                  srcv0, dstv0, adjv0, srcv1, dstv1, adjv1,
                  srcv2, dstv2, adjv2, srcv3, dstv3, adjv3,
                  rows0, rows1, rows2, rows3,
                  semi0, semi1, semi2, semi3,
                  semg0, semg1, semg2, semg3,
                  sems0, sems1, sems2, sems3):
    c = lax.axis_index("c")
    s = lax.axis_index("s")
    myk = jnp.where(c == 0, K0, K1)
    ebase = jnp.where(c == 0, s * K0, (NS * K0) + s * K1) * FCHUNK

    srcv = [srcv0, srcv1, srcv2, srcv3]
    dstv = [dstv0, dstv1, dstv2, dstv3]
    adjv = [adjv0, adjv1, adjv2, adjv3]
    rows = [rows0, rows1, rows2, rows3]
    semi = [semi0, semi1, semi2, semi3]
    semg = [semg0, semg1, semg2, semg3]
    sems = [sems0, sems1, sems2, sems3]

    def idx_start(ci, b):
        base = ebase + ci * FCHUNK
        pltpu.async_copy(src_hbm.at[pl.ds(base, FCHUNK)], srcv[b], semi[b])
        pltpu.async_copy(dst_hbm.at[pl.ds(base, FCHUNK)], dstv[b], semi[b])
        pltpu.async_copy(adj_hbm.at[pl.ds(base, FCHUNK)], adjv[b], semi[b])

    def idx_wait(b):
        pltpu.make_async_copy(src_hbm.at[pl.ds(0, FCHUNK)], srcv[b], semi[b]).wait()
        pltpu.make_async_copy(dst_hbm.at[pl.ds(0, FCHUNK)], dstv[b], semi[b]).wait()
        pltpu.make_async_copy(adj_hbm.at[pl.ds(0, FCHUNK)], adjv[b], semi[b]).wait()

    def gather_start(b):
        pltpu.async_copy(h_hbm.at[srcv[b]], rows[b], semg[b])

    def gather_wait(b):
        pltpu.make_async_copy(h_hbm.at[srcv[b]], rows[b], semg[b]).wait()

    def scatter_start(b):
        pltpu.async_copy(rows[b], acc_sh.at[dstv[b]], sems[b], add=True)

    def scatter_wait(b):
        pltpu.make_async_copy(rows[b], acc_sh.at[dstv[b]], sems[b]).wait()

    def scale(b):
        def _scale(g, _):
            av = adjv[b][pl.ds(g * 16, 16)]
            for j in range(16):
                a = jnp.full((16,), av[j])
                e = g * 16 + j
                for k in range(D // 16):
                    rows[b][e, pl.ds(k * 16, 16)] = (
                        rows[b][e, pl.ds(k * 16, 16)] * a)
            return 0
        lax.fori_loop(0, FCHUNK // 16, _scale, 0)

    z16 = jnp.zeros((16,), jnp.float32)

    def _zero_bufs(r, _):
        for k in range(D // 16):
            for b in range(NBUF):
                rows[b][r, pl.ds(k * 16, 16)] = z16
        return 0
    lax.fori_loop(0, FCHUNK, _zero_bufs, 0)

    row0 = s * ROWS_PER_TILE
    for k in range(ROWS_PER_TILE // FCHUNK):
        pltpu.sync_copy(rows0, acc_sh.at[pl.ds(row0 + k * FCHUNK, FCHUNK)])

    plsc.subcore_barrier()

    for b in range(NBUF):
        idx_start(b, b)
    for b in range(NBUF):
        idx_wait(b)
        gather_start(b)

    def _quad(k, _):
        for b in range(NBUF):
            cb = NBUF * k + b
            gather_wait(b)

            pb = (b + NBUF - 1) % NBUF

            @pl.when(cb >= 1)
            def _():
                scatter_wait(pb)

                @pl.when(cb + NBUF - 1 < myk)
                def _():
                    idx_start(cb + NBUF - 1, pb)
                    idx_wait(pb)
                    gather_start(pb)

            scale(b)
            scatter_start(b)
        return 0

    lax.fori_loop(0, myk // NBUF, _quad, 0)
    scatter_wait(NBUF - 1)

    plsc.subcore_barrier()

    for k in range(ROWS_PER_TILE // FCHUNK):
        r = row0 + k * FCHUNK
        pltpu.sync_copy(acc_sh.at[pl.ds(r, FCHUNK)], rows0)
        pltpu.sync_copy(rows0, acc_out.at[c, pl.ds(r, FCHUNK)])


def _sc_feat(h, src, dst, adj):
    mesh = plsc.VectorSubcoreMesh(core_axis_name="c", subcore_axis_name="s")
    idx_t = [pltpu.VMEM((FCHUNK,), jnp.int32),
             pltpu.VMEM((FCHUNK,), jnp.int32),
             pltpu.VMEM((FCHUNK,), jnp.float32)]
    f = pl.kernel(
        _sc_feat_body,
        out_type=jax.ShapeDtypeStruct((NC, N_PAD, D), jnp.float32),
        mesh=mesh,
        scratch_types=(
            [pltpu.VMEM_SHARED((N_PAD, D), jnp.float32)]
            + idx_t * NBUF
            + [pltpu.VMEM((FCHUNK, D), jnp.float32)] * NBUF
            + [pltpu.SemaphoreType.DMA] * (3 * NBUF)
        ),
    )
    return f(h, src, dst, adj)


def _sc_norm_body(dst_hbm, adj_hbm, norm_out, norm_sh,
                  dstv0, adjv0, dstv1, adjv1, nr0, nr1,
                  semi0, semi1, sems0, sems1):
    c = lax.axis_index("c")
    s = lax.axis_index("s")
    wid = c * NS + s
    ebase = wid * EPW

    def idx_start(ci, dv, av, sem):
        base = ebase + ci * FCHUNK
        pltpu.async_copy(dst_hbm.at[pl.ds(base, FCHUNK)], dv, sem)
        pltpu.async_copy(adj_hbm.at[pl.ds(base, FCHUNK)], av, sem)

    def idx_wait(dv, av, sem):
        pltpu.make_async_copy(dst_hbm.at[pl.ds(0, FCHUNK)], dv, sem).wait()
        pltpu.make_async_copy(adj_hbm.at[pl.ds(0, FCHUNK)], av, sem).wait()

    def fill(nr, av_ref):
        def _f(g, _):
            av = av_ref[pl.ds(g * 16, 16)]
            for j in range(16):
                nr[g * 16 + j, pl.ds(0, 16)] = jnp.full((16,), av[j])
            return 0
        lax.fori_loop(0, FCHUNK // 16, _f, 0)

    z16 = jnp.zeros((16,), jnp.float32)

    def _zero_bufs(r, _):
        for k in range(D // 16):
            nr0[r, pl.ds(k * 16, 16)] = z16
            nr1[r, pl.ds(k * 16, 16)] = z16
        return 0
    lax.fori_loop(0, FCHUNK, _zero_bufs, 0)

    row0 = s * ROWS_PER_TILE
    for k in range(ROWS_PER_TILE // FCHUNK):
        pltpu.sync_copy(nr0, norm_sh.at[pl.ds(row0 + k * FCHUNK, FCHUNK)])

    plsc.subcore_barrier()

    idx_start(0, dstv0, adjv0, semi0)
    idx_wait(dstv0, adjv0, semi0)
    fill(nr0, adjv0)
    idx_start(1, dstv1, adjv1, semi1)

    def _pair(k, _):
        c0 = 2 * k
        d0 = pltpu.async_copy(nr0, norm_sh.at[dstv0], sems0, add=True)
        idx_wait(dstv1, adjv1, semi1)
        fill(nr1, adjv1)
        d1 = pltpu.async_copy(nr1, norm_sh.at[dstv1], sems1, add=True)
        d0.wait()

        @pl.when(c0 + 2 < FN_CHUNKS)
        def _():
            idx_start(c0 + 2, dstv0, adjv0, semi0)
            idx_wait(dstv0, adjv0, semi0)
            fill(nr0, adjv0)

        d1.wait()

        @pl.when(c0 + 3 < FN_CHUNKS)
        def _():
            idx_start(c0 + 3, dstv1, adjv1, semi1)
        return 0

    lax.fori_loop(0, FN_CHUNKS // 2, _pair, 0)

    plsc.subcore_barrier()

    for k in range(ROWS_PER_TILE // FCHUNK):
        r = row0 + k * FCHUNK
        pltpu.sync_copy(norm_sh.at[pl.ds(r, FCHUNK)], nr0)
        pltpu.sync_copy(nr0, norm_out.at[c, pl.ds(r, FCHUNK)])


def _sc_norm(dst, adj):
    mesh = plsc.VectorSubcoreMesh(core_axis_name="c", subcore_axis_name="s")
    f = pl.kernel(
        _sc_norm_body,
        out_type=jax.ShapeDtypeStruct((NC, N_PAD, D), jnp.float32),
        mesh=mesh,
        scratch_types=[
            pltpu.VMEM_SHARED((N_PAD, D), jnp.float32),
            pltpu.VMEM((FCHUNK,), jnp.int32),
            pltpu.VMEM((FCHUNK,), jnp.float32),
            pltpu.VMEM((FCHUNK,), jnp.int32),
            pltpu.VMEM((FCHUNK,), jnp.float32),
            pltpu.VMEM((FCHUNK, D), jnp.float32),
            pltpu.VMEM((FCHUNK, D), jnp.float32),
            pltpu.SemaphoreType.DMA,
            pltpu.SemaphoreType.DMA,
            pltpu.SemaphoreType.DMA,
            pltpu.SemaphoreType.DMA,
        ],
    )
    return f(dst, adj)


def kernel(node_feat, edge_index, adj_values, W, b, W_self, b_self):
    x = node_feat[0]
    dst = edge_index[0].astype(jnp.int32)
    src = edge_index[1].astype(jnp.int32)
    pad = E_PAD - N_EDGES
    src_p = jnp.concatenate([src, jnp.zeros((pad,), jnp.int32)])
    dst_p = jnp.concatenate([dst, jnp.zeros((pad,), jnp.int32)])
    adj_p = jnp.concatenate([adj_values, jnp.zeros((pad,), jnp.float32)])
    x_pad = jnp.concatenate(
        [x, jnp.zeros((N_PAD - N_NODES, D), jnp.float32)], axis=0)

    h = _tc_linear(x, W, b)
    acc = _sc_feat(h, src_p, dst_p, adj_p)
    normp = _sc_norm(dst_p, adj_p)
    out = _tc_combine(acc[0], acc[1], normp[0], normp[1], x_pad, W_self, b_self)
    return out[:N_NODES][None]


# speedup vs baseline: 1.2300x; 1.0076x over previous
import jax
import jax.numpy as jnp
from jax import lax
from jax.experimental import pallas as pl
from jax.experimental.pallas import tpu as pltpu
from jax.experimental.pallas import tpu_sc as plsc

N_NODES = 10000
D = 128
N_EDGES = 320000

NC = 2
NS = 16
NW = NC * NS
CHUNK = 128
FCHUNK = 32
FN_CHUNKS = 10240 // FCHUNK
K0 = 400
K1 = 240
E_PAD = 327680
EPW = E_PAD // NW
N_CHUNKS = EPW // CHUNK
N_PAD = 10240
ROWS_PER_TILE = N_PAD // NS
ZROWS = 128
BLK = 1024


def _linear_body(x_ref, w_ref, b_ref, o_ref):
    o_ref[...] = lax.dot_general(
        x_ref[...], w_ref[...], (((1,), (1,)), ((), ())),
        preferred_element_type=jnp.float32) + b_ref[...]


def _tc_linear(x, W, b):
    return pl.pallas_call(
        _linear_body,
        grid=(10,),
        in_specs=[
            pl.BlockSpec((1000, D), lambda i: (i, 0)),
            pl.BlockSpec((D, D), lambda i: (0, 0)),
            pl.BlockSpec((1, D), lambda i: (0, 0)),
        ],
        out_specs=pl.BlockSpec((1000, D), lambda i: (i, 0)),
        out_shape=jax.ShapeDtypeStruct((N_NODES, D), jnp.float32),
    )(x, W, b.reshape(1, D))


def _combine_body(a0_ref, a1_ref, n0_ref, n1_ref, x_ref, w_ref, b_ref, o_ref):
    norm = n0_ref[...][:, 0:1] + n1_ref[...][:, 0:1]
    acc = a0_ref[...] + a1_ref[...]
    selfh = lax.dot_general(
        x_ref[...], w_ref[...], (((1,), (1,)), ((), ())),
        preferred_element_type=jnp.float32) + b_ref[...]
    o_ref[...] = acc / norm + selfh


def _tc_combine(a0, a1, n0, n1, x_pad, W_self, b_self):
    return pl.pallas_call(
        _combine_body,
        grid=(10,),
        in_specs=[
            pl.BlockSpec((BLK, D), lambda i: (i, 0)),
            pl.BlockSpec((BLK, D), lambda i: (i, 0)),
            pl.BlockSpec((BLK, D), lambda i: (i, 0)),
            pl.BlockSpec((BLK, D), lambda i: (i, 0)),
            pl.BlockSpec((BLK, D), lambda i: (i, 0)),
            pl.BlockSpec((D, D), lambda i: (0, 0)),
            pl.BlockSpec((1, D), lambda i: (0, 0)),
        ],
        out_specs=pl.BlockSpec((BLK, D), lambda i: (i, 0)),
        out_shape=jax.ShapeDtypeStruct((N_PAD, D), jnp.float32),
    )(a0, a1, n0, n1, x_pad, W_self, b_self.reshape(1, D))


NBUF = 4


def _sc_feat_body(h_hbm, src_hbm, dst_hbm, adj_hbm,
                  acc_out,
                  acc_sh,
                  srcv0, dstv0, adjv0, srcv1, dstv1, adjv1,
                  srcv2, dstv2, adjv2, srcv3, dstv3, adjv3,
                  rows0, rows1, rows2, rows3,
                  semi0, semi1, semi2, semi3,
                  semg0, semg1, semg2, semg3,
                  sems0, sems1, sems2, sems3):
    c = lax.axis_index("c")
    s = lax.axis_index("s")
    myk = jnp.where(c == 0, K0, K1)
    ebase = jnp.where(c == 0, s * K0, (NS * K0) + s * K1) * FCHUNK

    srcv = [srcv0, srcv1, srcv2, srcv3]
    dstv = [dstv0, dstv1, dstv2, dstv3]
    adjv = [adjv0, adjv1, adjv2, adjv3]
    rows = [rows0, rows1, rows2, rows3]
    semi = [semi0, semi1, semi2, semi3]
    semg = [semg0, semg1, semg2, semg3]
    sems = [sems0, sems1, sems2, sems3]

    def idx_start(ci, b):
        base = ebase + ci * FCHUNK
        pltpu.async_copy(src_hbm.at[pl.ds(base, FCHUNK)], srcv[b], semi[b])
        pltpu.async_copy(dst_hbm.at[pl.ds(base, FCHUNK)], dstv[b], semi[b])
        pltpu.async_copy(adj_hbm.at[pl.ds(base, FCHUNK)], adjv[b], semi[b])

    def idx_wait(b):
        pltpu.make_async_copy(src_hbm.at[pl.ds(0, FCHUNK)], srcv[b], semi[b]).wait()
        pltpu.make_async_copy(dst_hbm.at[pl.ds(0, FCHUNK)], dstv[b], semi[b]).wait()
        pltpu.make_async_copy(adj_hbm.at[pl.ds(0, FCHUNK)], adjv[b], semi[b]).wait()

    def gather_start(b):
        pltpu.async_copy(h_hbm.at[srcv[b]], rows[b], semg[b])

    def gather_wait(b):
        pltpu.make_async_copy(h_hbm.at[srcv[b]], rows[b], semg[b]).wait()

    def scatter_start(b):
        pltpu.async_copy(rows[b], acc_sh.at[dstv[b]], sems[b], add=True)

    def scatter_wait(b):
        pltpu.make_async_copy(rows[b], acc_sh.at[dstv[b]], sems[b]).wait()

    def scale(b):
        def _scale(g, _):
            av = adjv[b][pl.ds(g * 16, 16)]
            for j in range(16):
                a = jnp.full((16,), av[j])
                e = g * 16 + j
                for k in range(D // 16):
                    rows[b][e, pl.ds(k * 16, 16)] = (
                        rows[b][e, pl.ds(k * 16, 16)] * a)
            return 0
        lax.fori_loop(0, FCHUNK // 16, _scale, 0)

    z16 = jnp.zeros((16,), jnp.float32)

    def _zero_bufs(r, _):
        for k in range(D // 16):
            for b in range(NBUF):
                rows[b][r, pl.ds(k * 16, 16)] = z16
        return 0
    lax.fori_loop(0, FCHUNK, _zero_bufs, 0)

    row0 = s * ROWS_PER_TILE
    for k in range(ROWS_PER_TILE // FCHUNK):
        pltpu.sync_copy(rows0, acc_sh.at[pl.ds(row0 + k * FCHUNK, FCHUNK)])

    plsc.subcore_barrier()

    for b in range(NBUF):
        idx_start(b, b)
    for b in range(NBUF):
        idx_wait(b)
        gather_start(b)

    def _quad(k, _):
        for b in range(NBUF):
            cb = NBUF * k + b
            gather_wait(b)

            pb = (b + NBUF - 1) % NBUF

            @pl.when(cb >= 1)
            def _():
                scatter_wait(pb)

                @pl.when(cb + NBUF - 1 < myk)
                def _():
                    idx_start(cb + NBUF - 1, pb)
                    idx_wait(pb)
                    gather_start(pb)

            scale(b)
            scatter_start(b)
        return 0

    lax.fori_loop(0, myk // NBUF, _quad, 0)
    scatter_wait(NBUF - 1)

    plsc.subcore_barrier()

    for k in range(ROWS_PER_TILE // FCHUNK):
        r = row0 + k * FCHUNK
        pltpu.sync_copy(acc_sh.at[pl.ds(r, FCHUNK)], rows0)
        pltpu.sync_copy(rows0, acc_out.at[c, pl.ds(r, FCHUNK)])


def _sc_feat(h, src, dst, adj):
    mesh = plsc.VectorSubcoreMesh(core_axis_name="c", subcore_axis_name="s")
    idx_t = [pltpu.VMEM((FCHUNK,), jnp.int32),
             pltpu.VMEM((FCHUNK,), jnp.int32),
             pltpu.VMEM((FCHUNK,), jnp.float32)]
    f = pl.kernel(
        _sc_feat_body,
        out_type=jax.ShapeDtypeStruct((NC, N_PAD, D), jnp.float32),
        mesh=mesh,
        scratch_types=(
            [pltpu.VMEM_SHARED((N_PAD, D), jnp.float32)]
            + idx_t * NBUF
            + [pltpu.VMEM((FCHUNK, D), jnp.float32)] * NBUF
            + [pltpu.SemaphoreType.DMA] * (3 * NBUF)
        ),
    )
    return f(h, src, dst, adj)


def _sc_norm_body(dst_hbm, adj_hbm, norm_out, norm_sh,
                  dstv0, adjv0, dstv1, adjv1, nr0, nr1,
                  semi0, semi1, sems0, sems1):
    c = lax.axis_index("c")
    s = lax.axis_index("s")
    wid = c * NS + s
    ebase = wid * EPW

    def idx_start(ci, dv, av, sem):
        base = ebase + ci * FCHUNK
        pltpu.async_copy(dst_hbm.at[pl.ds(base, FCHUNK)], dv, sem)
        pltpu.async_copy(adj_hbm.at[pl.ds(base, FCHUNK)], av, sem)

    def idx_wait(dv, av, sem):
        pltpu.make_async_copy(dst_hbm.at[pl.ds(0, FCHUNK)], dv, sem).wait()
        pltpu.make_async_copy(adj_hbm.at[pl.ds(0, FCHUNK)], av, sem).wait()

    def fill(nr, av_ref):
        def _f(g, _):
            av = av_ref[pl.ds(g * 16, 16)]
            for j in range(16):
                nr[g * 16 + j, pl.ds(0, 16)] = jnp.full((16,), av[j])
            return 0
        lax.fori_loop(0, FCHUNK // 16, _f, 0)

    z16 = jnp.zeros((16,), jnp.float32)

    def _zero_bufs(r, _):
        for k in range(D // 16):
            nr0[r, pl.ds(k * 16, 16)] = z16
            nr1[r, pl.ds(k * 16, 16)] = z16
        return 0
    lax.fori_loop(0, FCHUNK, _zero_bufs, 0)

    row0 = s * ROWS_PER_TILE
    for k in range(ROWS_PER_TILE // FCHUNK):
        pltpu.sync_copy(nr0, norm_sh.at[pl.ds(row0 + k * FCHUNK, FCHUNK)])

    plsc.subcore_barrier()

    idx_start(0, dstv0, adjv0, semi0)
    idx_wait(dstv0, adjv0, semi0)
    fill(nr0, adjv0)
    idx_start(1, dstv1, adjv1, semi1)

    def _pair(k, _):
        c0 = 2 * k
        d0 = pltpu.async_copy(nr0, norm_sh.at[dstv0], sems0, add=True)
        idx_wait(dstv1, adjv1, semi1)
        fill(nr1, adjv1)
        d1 = pltpu.async_copy(nr1, norm_sh.at[dstv1], sems1, add=True)
        d0.wait()

        @pl.when(c0 + 2 < FN_CHUNKS)
        def _():
            idx_start(c0 + 2, dstv0, adjv0, semi0)
            idx_wait(dstv0, adjv0, semi0)
            fill(nr0, adjv0)

        d1.wait()

        @pl.when(c0 + 3 < FN_CHUNKS)
        def _():
            idx_start(c0 + 3, dstv1, adjv1, semi1)
        return 0

    lax.fori_loop(0, FN_CHUNKS // 2, _pair, 0)

    plsc.subcore_barrier()

    for k in range(ROWS_PER_TILE // FCHUNK):
        r = row0 + k * FCHUNK
        pltpu.sync_copy(norm_sh.at[pl.ds(r, FCHUNK)], nr0)
        pltpu.sync_copy(nr0, norm_out.at[c, pl.ds(r, FCHUNK)])


def _sc_norm(dst, adj):
    mesh = plsc.VectorSubcoreMesh(core_axis_name="c", subcore_axis_name="s")
    f = pl.kernel(
        _sc_norm_body,
        out_type=jax.ShapeDtypeStruct((NC, N_PAD, D), jnp.float32),
        mesh=mesh,
        scratch_types=[
            pltpu.VMEM_SHARED((N_PAD, D), jnp.float32),
            pltpu.VMEM((FCHUNK,), jnp.int32),
            pltpu.VMEM((FCHUNK,), jnp.float32),
            pltpu.VMEM((FCHUNK,), jnp.int32),
            pltpu.VMEM((FCHUNK,), jnp.float32),
            pltpu.VMEM((FCHUNK, D), jnp.float32),
            pltpu.VMEM((FCHUNK, D), jnp.float32),
            pltpu.SemaphoreType.DMA,
            pltpu.SemaphoreType.DMA,
            pltpu.SemaphoreType.DMA,
            pltpu.SemaphoreType.DMA,
        ],
    )
    return f(dst, adj)


def kernel(node_feat, edge_index, adj_values, W, b, W_self, b_self):
    x = node_feat[0]
    dst = edge_index[0].astype(jnp.int32)
    src = edge_index[1].astype(jnp.int32)
    pad = E_PAD - N_EDGES
    src_p = jnp.concatenate([src, jnp.zeros((pad,), jnp.int32)])
    dst_p = jnp.concatenate([dst, jnp.zeros((pad,), jnp.int32)])
    adj_p = jnp.concatenate([adj_values, jnp.zeros((pad,), jnp.float32)])
    x_pad = jnp.concatenate(
        [x, jnp.zeros((N_PAD - N_NODES, D), jnp.float32)], axis=0)

    h = _tc_linear(x, W, b)
    acc = _sc_feat(h, src_p, dst_p, adj_p)
    normp = _sc_norm(dst_p, adj_p)
    out = _tc_combine(acc[0], acc[1], normp[0], normp[1], x_pad, W_self, b_self)
    return out[:N_NODES][None]
